# baseline (speedup 1.0000x reference)
import functools

import jax
import jax.numpy as jnp
from jax import lax
from jax.experimental import pallas as pl
from jax.experimental.pallas import tpu as pltpu
from jax.experimental.pallas import tpu_sc as plsc

N = 10000
E = 320000
D = 128

NC = 2
NS = 16
NW = NC * NS
EPW = E // NW
CH = 80
NCHUNK = EPW // CH
ROWS_A = 632
ROWS_LAST = N - 15 * ROWS_A

_mesh = plsc.VectorSubcoreMesh(core_axis_name="c", subcore_axis_name="s")


def _build_idx(slab_v, k, buf_v):
    for j in range(CH // 16):
        buf_v[pl.ds(j * 16, 16)] = slab_v[pl.ds(k * CH + j * 16, 16)]


@functools.partial(
    pl.kernel,
    out_type=jax.ShapeDtypeStruct((NC * N,), jnp.float32),
    mesh=_mesh,
    scratch_types=[
        pltpu.VMEM((EPW,), jnp.int32),
        pltpu.VMEM((EPW,), jnp.float32),
        pltpu.VMEM((CH,), jnp.int32),
        pltpu.VMEM((CH,), jnp.int32),
        pltpu.VMEM((N,), jnp.float32),
        pltpu.VMEM_SHARED((N,), jnp.float32),
        pltpu.SemaphoreType.DMA,
        pltpu.SemaphoreType.DMA,
    ],
)
def _deg_kernel(ei_hbm, ew_hbm, out_hbm, dsts_v, w_v, idx_a, idx_b,
                stage_v, deg_sh, sem_a, sem_b):
    c = lax.axis_index("c")
    s = lax.axis_index("s")
    wid = c * NS + s

    pltpu.async_copy(ei_hbm.at[pl.ds(E + wid * EPW, EPW)], dsts_v, sem_a)
    pltpu.async_copy(ew_hbm.at[pl.ds(wid * EPW, EPW)], w_v, sem_b)

    @pl.when(s == 0)
    def _():
        def z(k, carry):
            stage_v[pl.ds(k * 16, 16)] = jnp.zeros((16,), jnp.float32)
            return carry

        lax.fori_loop(0, N // 16, z, 0)
        pltpu.sync_copy(stage_v, deg_sh)

    pltpu.make_async_copy(ei_hbm.at[pl.ds(E + wid * EPW, EPW)], dsts_v,
                          sem_a).wait()
    pltpu.make_async_copy(ew_hbm.at[pl.ds(wid * EPW, EPW)], w_v,
                          sem_b).wait()
    plsc.subcore_barrier()

    def fire(k, idx_v, sem):
        pltpu.async_copy(w_v.at[pl.ds(k * CH, CH)], deg_sh.at[idx_v], sem,
                         add=True)

    def wait(k, idx_v, sem):
        pltpu.make_async_copy(w_v.at[pl.ds(k * CH, CH)], deg_sh.at[idx_v],
                              sem).wait()

    _build_idx(dsts_v, 0, idx_a)
    fire(0, idx_a, sem_a)

    def pipe(i, carry):
        k = 2 * i
        _build_idx(dsts_v, k + 1, idx_b)

        @pl.when(i > 0)
        def _():
            wait(k - 1, idx_b, sem_b)

        fire(k + 1, idx_b, sem_b)
        wait(k, idx_a, sem_a)

        @pl.when(k + 2 < NCHUNK)
        def _():
            _build_idx(dsts_v, k + 2, idx_a)
            fire(k + 2, idx_a, sem_a)

        return carry

    lax.fori_loop(0, NCHUNK // 2, pipe, 0)
    wait(NCHUNK - 2, idx_b, sem_b)
    wait(NCHUNK - 1, idx_a, sem_a)
    plsc.subcore_barrier()

    @pl.when(s == 0)
    def _():
        pltpu.sync_copy(deg_sh, stage_v)
        pltpu.sync_copy(stage_v, out_hbm.at[pl.ds(c * N, N)])


def _mmg_body(x_ref, w_ref, degt_ref, g_ref, dis_ref):
    h = jnp.dot(x_ref[...], w_ref[...], preferred_element_type=jnp.float32)
    dis = lax.rsqrt(jnp.sum(degt_ref[...], axis=1, keepdims=True) + 1.0)
    dis_ref[...] = dis
    g_ref[...] = h * dis


def _mmg(x, W, degt):
    R = 1000
    return pl.pallas_call(
        _mmg_body,
        out_shape=(jax.ShapeDtypeStruct((N, D), jnp.float32),
                   jax.ShapeDtypeStruct((N, 1), jnp.float32)),
        grid=(N // R,),
        in_specs=[
            pl.BlockSpec((R, D), lambda i: (i, 0)),
            pl.BlockSpec((D, D), lambda i: (0, 0)),
            pl.BlockSpec((R, NC), lambda i: (i, 0)),
        ],
        out_specs=(pl.BlockSpec((R, D), lambda i: (i, 0)),
                   pl.BlockSpec((R, 1), lambda i: (i, 0))),
    )(x, W, degt)


@functools.partial(
    pl.kernel,
    out_type=jax.ShapeDtypeStruct((NC, N, D), jnp.float32),
    mesh=_mesh,
    scratch_types=[
        pltpu.VMEM((EPW,), jnp.int32),
        pltpu.VMEM((EPW,), jnp.int32),
        pltpu.VMEM((EPW,), jnp.float32),
        pltpu.VMEM((CH,), jnp.int32),
        pltpu.VMEM((CH,), jnp.int32),
        pltpu.VMEM((CH, D), jnp.float32),
        pltpu.VMEM((CH, D), jnp.float32),
        pltpu.VMEM_SHARED((N, D), jnp.float32),
        pltpu.SemaphoreType.DMA,
        pltpu.SemaphoreType.DMA,
        pltpu.SemaphoreType.DMA,
        pltpu.SemaphoreType.DMA,
    ],
)
def _agg_kernel(g_hbm, ei_hbm, ew_hbm, out_hbm,
                src_v, dsts_v, w_v, dst_a, dst_b, rows_a, rows_b,
                acc_sh, sem_ga, sem_gb, sem_sa, sem_sb):
    c = lax.axis_index("c")
    s = lax.axis_index("s")
    wid = c * NS + s

    pltpu.async_copy(ei_hbm.at[pl.ds(wid * EPW, EPW)], src_v, sem_ga)
    pltpu.async_copy(ei_hbm.at[pl.ds(E + wid * EPW, EPW)], dsts_v, sem_gb)
    pltpu.async_copy(ew_hbm.at[pl.ds(wid * EPW, EPW)], w_v, sem_sb)

    def zrow(e, carry):
        for cc in range(D // 16):
            rows_a[e, pl.ds(cc * 16, 16)] = jnp.zeros((16,), jnp.float32)
        return carry

    lax.fori_loop(0, CH, zrow, 0)
    zbase = s * (N // NS)
    for k in range(7):
        pltpu.async_copy(rows_a, acc_sh.at[pl.ds(zbase + k * CH, CH)],
                         sem_sa)
    for k in range(7):
        pltpu.make_async_copy(rows_a, acc_sh.at[pl.ds(zbase + k * CH, CH)],
                              sem_sa).wait()
    pltpu.sync_copy(rows_a.at[pl.ds(0, 65)],
                    acc_sh.at[pl.ds(zbase + 7 * CH, 65)])

    pltpu.make_async_copy(ei_hbm.at[pl.ds(wid * EPW, EPW)], src_v,
                          sem_ga).wait()
    pltpu.make_async_copy(ei_hbm.at[pl.ds(E + wid * EPW, EPW)], dsts_v,
                          sem_gb).wait()
    pltpu.make_async_copy(ew_hbm.at[pl.ds(wid * EPW, EPW)], w_v,
                          sem_sb).wait()
    plsc.subcore_barrier()

    def fire_gather(k, rows, sem):
        pltpu.async_copy(g_hbm.at[src_v.at[pl.ds(k * CH, CH)]], rows, sem)

    def wait_gather(k, rows, sem):
        pltpu.make_async_copy(g_hbm.at[src_v.at[pl.ds(k * CH, CH)]], rows,
                              sem).wait()

    def fire_scatter(rows, dst_v, sem):
        pltpu.async_copy(rows, acc_sh.at[dst_v], sem, add=True)

    def wait_scatter(rows, dst_v, sem):
        pltpu.make_async_copy(rows, acc_sh.at[dst_v], sem).wait()

    def scale(k, rows):
        def scale_grp(g, carry2):
            sv16 = w_v[pl.ds(k * CH + g * 16, 16)]
            for j in range(16):
                e = g * 16 + j
                se = sv16[j]
                for cc in range(D // 16):
                    sl = pl.ds(cc * 16, 16)
                    rows[e, sl] = rows[e, sl] * se
            return carry2

        lax.fori_loop(0, CH // 16, scale_grp, 0)

    fire_gather(0, rows_a, sem_ga)

    def pipe(i, carry):
        k = 2 * i
        wait_gather(k, rows_a, sem_ga)

        @pl.when(i > 0)
        def _():
            wait_scatter(rows_b, dst_b, sem_sb)

        fire_gather(k + 1, rows_b, sem_gb)
        scale(k, rows_a)
        _build_idx(dsts_v, k, dst_a)
        fire_scatter(rows_a, dst_a, sem_sa)

        wait_gather(k + 1, rows_b, sem_gb)

        @pl.when(k + 2 < NCHUNK)
        def _():
            wait_scatter(rows_a, dst_a, sem_sa)
            fire_gather(k + 2, rows_a, sem_ga)

        scale(k + 1, rows_b)
        _build_idx(dsts_v, k + 1, dst_b)
        fire_scatter(rows_b, dst_b, sem_sb)
        return carry

    lax.fori_loop(0, NCHUNK // 2, pipe, 0)

    k_last = NCHUNK - 1
    wait_gather(k_last, rows_a, sem_ga)
    wait_scatter(rows_b, dst_b, sem_sb)
    scale(k_last, rows_a)
    _build_idx(dsts_v, k_last, dst_a)
    fire_scatter(rows_a, dst_a, sem_sa)
    wait_scatter(rows_a, dst_a, sem_sa)
    plsc.subcore_barrier()

    @pl.when(s < NS - 1)
    def _():
        rb = s * ROWS_A
        pltpu.sync_copy(acc_sh.at[pl.ds(rb, ROWS_A)],
                        out_hbm.at[c, pl.ds(rb, ROWS_A)])

    @pl.when(s == NS - 1)
    def _():
        rb = (NS - 1) * ROWS_A
        pltpu.sync_copy(acc_sh.at[pl.ds(rb, ROWS_LAST)],
                        out_hbm.at[c, pl.ds(rb, ROWS_LAST)])


def _fin_body(acc0_ref, acc1_ref, g_ref, dis_ref, b_ref, out_ref):
    dis = dis_ref[...]
    o = dis * (acc0_ref[0] + acc1_ref[0] + g_ref[...])
    out_ref[...] = jnp.maximum(o + b_ref[...], 0.0)


def _finalize(acc, g, dist, b):
    R = 1000
    return pl.pallas_call(
        _fin_body,
        out_shape=jax.ShapeDtypeStruct((N, D), jnp.float32),
        grid=(N // R,),
        in_specs=[
            pl.BlockSpec((1, R, D), lambda i: (0, i, 0)),
            pl.BlockSpec((1, R, D), lambda i: (1, i, 0)),
            pl.BlockSpec((R, D), lambda i: (i, 0)),
            pl.BlockSpec((R, 1), lambda i: (i, 0)),
            pl.BlockSpec((1, D), lambda i: (0, 0)),
        ],
        out_specs=pl.BlockSpec((R, D), lambda i: (i, 0)),
    )(acc, acc, g, dist, b.reshape(1, D))


def kernel(x, edge_index, edge_weights, W, b):
    ei_flat = edge_index.reshape(2 * E)
    degp = _deg_kernel(ei_flat, edge_weights)
    g, dis = _mmg(x, W, degp.reshape(NC, N).T)
    acc = _agg_kernel(g, ei_flat, edge_weights)
    return _finalize(acc, g, dis, b)


# speedup vs baseline: 1.0020x; 1.0020x over previous
import functools

import jax
import jax.numpy as jnp
from jax import lax
from jax.experimental import pallas as pl
from jax.experimental.pallas import tpu as pltpu
from jax.experimental.pallas import tpu_sc as plsc

N = 10000
E = 320000
D = 128

NC = 2
NS = 16
NW = NC * NS
EPW = E // NW
CH = 80
NCHUNK = EPW // CH
ROWS_A = 632
ROWS_LAST = N - 15 * ROWS_A

_mesh = plsc.VectorSubcoreMesh(core_axis_name="c", subcore_axis_name="s")


def _build_idx(slab_v, k, buf_v):
    for j in range(CH // 16):
        buf_v[pl.ds(j * 16, 16)] = slab_v[pl.ds(k * CH + j * 16, 16)]


@functools.partial(
    pl.kernel,
    out_type=jax.ShapeDtypeStruct((NC * N,), jnp.float32),
    mesh=_mesh,
    compiler_params=pltpu.CompilerParams(needs_layout_passes=False),
    scratch_types=[
        pltpu.VMEM((EPW,), jnp.int32),
        pltpu.VMEM((EPW,), jnp.float32),
        pltpu.VMEM((CH,), jnp.int32),
        pltpu.VMEM((CH,), jnp.int32),
        pltpu.VMEM((N,), jnp.float32),
        pltpu.VMEM_SHARED((N,), jnp.float32),
        pltpu.SemaphoreType.DMA,
        pltpu.SemaphoreType.DMA,
    ],
)
def _deg_kernel(ei_hbm, ew_hbm, out_hbm, dsts_v, w_v, idx_a, idx_b,
                stage_v, deg_sh, sem_a, sem_b):
    c = lax.axis_index("c")
    s = lax.axis_index("s")
    wid = c * NS + s

    pltpu.async_copy(ei_hbm.at[pl.ds(E + wid * EPW, EPW)], dsts_v, sem_a)
    pltpu.async_copy(ew_hbm.at[pl.ds(wid * EPW, EPW)], w_v, sem_b)

    @pl.when(s == 0)
    def _():
        def z(k, carry):
            stage_v[pl.ds(k * 16, 16)] = jnp.zeros((16,), jnp.float32)
            return carry

        lax.fori_loop(0, N // 16, z, 0)
        pltpu.sync_copy(stage_v, deg_sh)

    pltpu.make_async_copy(ei_hbm.at[pl.ds(E + wid * EPW, EPW)], dsts_v,
                          sem_a).wait()
    pltpu.make_async_copy(ew_hbm.at[pl.ds(wid * EPW, EPW)], w_v,
                          sem_b).wait()
    plsc.subcore_barrier()

    def fire(k, idx_v, sem):
        pltpu.async_copy(w_v.at[pl.ds(k * CH, CH)], deg_sh.at[idx_v], sem,
                         add=True)

    def wait(k, idx_v, sem):
        pltpu.make_async_copy(w_v.at[pl.ds(k * CH, CH)], deg_sh.at[idx_v],
                              sem).wait()

    _build_idx(dsts_v, 0, idx_a)
    fire(0, idx_a, sem_a)

    def pipe(i, carry):
        k = 2 * i
        _build_idx(dsts_v, k + 1, idx_b)

        @pl.when(i > 0)
        def _():
            wait(k - 1, idx_b, sem_b)

        fire(k + 1, idx_b, sem_b)
        wait(k, idx_a, sem_a)

        @pl.when(k + 2 < NCHUNK)
        def _():
            _build_idx(dsts_v, k + 2, idx_a)
            fire(k + 2, idx_a, sem_a)

        return carry

    lax.fori_loop(0, NCHUNK // 2, pipe, 0)
    wait(NCHUNK - 2, idx_b, sem_b)
    wait(NCHUNK - 1, idx_a, sem_a)
    plsc.subcore_barrier()

    @pl.when(s == 0)
    def _():
        pltpu.sync_copy(deg_sh, stage_v)
        pltpu.sync_copy(stage_v, out_hbm.at[pl.ds(c * N, N)])


def _mmg_body(x_ref, w_ref, degt_ref, g_ref, dis_ref):
    h = jnp.dot(x_ref[...], w_ref[...], preferred_element_type=jnp.float32)
    dis = lax.rsqrt(jnp.sum(degt_ref[...], axis=1, keepdims=True) + 1.0)
    dis_ref[...] = dis
    g_ref[...] = h * dis


def _mmg(x, W, degt):
    R = 1000
    return pl.pallas_call(
        _mmg_body,
        out_shape=(jax.ShapeDtypeStruct((N, D), jnp.float32),
                   jax.ShapeDtypeStruct((N, 1), jnp.float32)),
        grid=(N // R,),
        in_specs=[
            pl.BlockSpec((R, D), lambda i: (i, 0)),
            pl.BlockSpec((D, D), lambda i: (0, 0)),
            pl.BlockSpec((R, NC), lambda i: (i, 0)),
        ],
        out_specs=(pl.BlockSpec((R, D), lambda i: (i, 0)),
                   pl.BlockSpec((R, 1), lambda i: (i, 0))),
    )(x, W, degt)


@functools.partial(
    pl.kernel,
    out_type=jax.ShapeDtypeStruct((NC, N, D), jnp.float32),
    mesh=_mesh,
    compiler_params=pltpu.CompilerParams(needs_layout_passes=False),
    scratch_types=[
        pltpu.VMEM((EPW,), jnp.int32),
        pltpu.VMEM((EPW,), jnp.int32),
        pltpu.VMEM((EPW,), jnp.float32),
        pltpu.VMEM((CH,), jnp.int32),
        pltpu.VMEM((CH,), jnp.int32),
        pltpu.VMEM((CH, D), jnp.float32),
        pltpu.VMEM((CH, D), jnp.float32),
        pltpu.VMEM_SHARED((N, D), jnp.float32),
        pltpu.SemaphoreType.DMA,
        pltpu.SemaphoreType.DMA,
        pltpu.SemaphoreType.DMA,
        pltpu.SemaphoreType.DMA,
    ],
)
def _agg_kernel(g_hbm, ei_hbm, ew_hbm, out_hbm,
                src_v, dsts_v, w_v, dst_a, dst_b, rows_a, rows_b,
                acc_sh, sem_ga, sem_gb, sem_sa, sem_sb):
    c = lax.axis_index("c")
    s = lax.axis_index("s")
    wid = c * NS + s

    pltpu.async_copy(ei_hbm.at[pl.ds(wid * EPW, EPW)], src_v, sem_ga)
    pltpu.async_copy(ei_hbm.at[pl.ds(E + wid * EPW, EPW)], dsts_v, sem_gb)
    pltpu.async_copy(ew_hbm.at[pl.ds(wid * EPW, EPW)], w_v, sem_sb)

    def zrow(e, carry):
        for cc in range(D // 16):
            rows_a[e, pl.ds(cc * 16, 16)] = jnp.zeros((16,), jnp.float32)
        return carry

    lax.fori_loop(0, CH, zrow, 0)
    zbase = s * (N // NS)
    for k in range(7):
        pltpu.async_copy(rows_a, acc_sh.at[pl.ds(zbase + k * CH, CH)],
                         sem_sa)
    for k in range(7):
        pltpu.make_async_copy(rows_a, acc_sh.at[pl.ds(zbase + k * CH, CH)],
                              sem_sa).wait()
    pltpu.sync_copy(rows_a.at[pl.ds(0, 65)],
                    acc_sh.at[pl.ds(zbase + 7 * CH, 65)])

    pltpu.make_async_copy(ei_hbm.at[pl.ds(wid * EPW, EPW)], src_v,
                          sem_ga).wait()
    pltpu.make_async_copy(ei_hbm.at[pl.ds(E + wid * EPW, EPW)], dsts_v,
                          sem_gb).wait()
    pltpu.make_async_copy(ew_hbm.at[pl.ds(wid * EPW, EPW)], w_v,
                          sem_sb).wait()
    plsc.subcore_barrier()

    def fire_gather(k, rows, sem):
        pltpu.async_copy(g_hbm.at[src_v.at[pl.ds(k * CH, CH)]], rows, sem)

    def wait_gather(k, rows, sem):
        pltpu.make_async_copy(g_hbm.at[src_v.at[pl.ds(k * CH, CH)]], rows,
                              sem).wait()

    def fire_scatter(rows, dst_v, sem):
        pltpu.async_copy(rows, acc_sh.at[dst_v], sem, add=True)

    def wait_scatter(rows, dst_v, sem):
        pltpu.make_async_copy(rows, acc_sh.at[dst_v], sem).wait()

    def scale(k, rows):
        def scale_grp(g, carry2):
            sv16 = w_v[pl.ds(k * CH + g * 16, 16)]
            for j in range(16):
                e = g * 16 + j
                se = sv16[j]
                for cc in range(D // 16):
                    sl = pl.ds(cc * 16, 16)
                    rows[e, sl] = rows[e, sl] * se
            return carry2

        lax.fori_loop(0, CH // 16, scale_grp, 0)

    fire_gather(0, rows_a, sem_ga)

    def pipe(i, carry):
        k = 2 * i
        wait_gather(k, rows_a, sem_ga)

        @pl.when(i > 0)
        def _():
            wait_scatter(rows_b, dst_b, sem_sb)

        fire_gather(k + 1, rows_b, sem_gb)
        scale(k, rows_a)
        _build_idx(dsts_v, k, dst_a)
        fire_scatter(rows_a, dst_a, sem_sa)

        wait_gather(k + 1, rows_b, sem_gb)

        @pl.when(k + 2 < NCHUNK)
        def _():
            wait_scatter(rows_a, dst_a, sem_sa)
            fire_gather(k + 2, rows_a, sem_ga)

        scale(k + 1, rows_b)
        _build_idx(dsts_v, k + 1, dst_b)
        fire_scatter(rows_b, dst_b, sem_sb)
        return carry

    lax.fori_loop(0, NCHUNK // 2, pipe, 0)

    k_last = NCHUNK - 1
    wait_gather(k_last, rows_a, sem_ga)
    wait_scatter(rows_b, dst_b, sem_sb)
    scale(k_last, rows_a)
    _build_idx(dsts_v, k_last, dst_a)
    fire_scatter(rows_a, dst_a, sem_sa)
    wait_scatter(rows_a, dst_a, sem_sa)
    plsc.subcore_barrier()

    @pl.when(s < NS - 1)
    def _():
        rb = s * ROWS_A
        pltpu.sync_copy(acc_sh.at[pl.ds(rb, ROWS_A)],
                        out_hbm.at[c, pl.ds(rb, ROWS_A)])

    @pl.when(s == NS - 1)
    def _():
        rb = (NS - 1) * ROWS_A
        pltpu.sync_copy(acc_sh.at[pl.ds(rb, ROWS_LAST)],
                        out_hbm.at[c, pl.ds(rb, ROWS_LAST)])


def _fin_body(acc0_ref, acc1_ref, g_ref, dis_ref, b_ref, out_ref):
    dis = dis_ref[...]
    o = dis * (acc0_ref[0] + acc1_ref[0] + g_ref[...])
    out_ref[...] = jnp.maximum(o + b_ref[...], 0.0)


def _finalize(acc, g, dist, b):
    R = 1000
    return pl.pallas_call(
        _fin_body,
        out_shape=jax.ShapeDtypeStruct((N, D), jnp.float32),
        grid=(N // R,),
        in_specs=[
            pl.BlockSpec((1, R, D), lambda i: (0, i, 0)),
            pl.BlockSpec((1, R, D), lambda i: (1, i, 0)),
            pl.BlockSpec((R, D), lambda i: (i, 0)),
            pl.BlockSpec((R, 1), lambda i: (i, 0)),
            pl.BlockSpec((1, D), lambda i: (0, 0)),
        ],
        out_specs=pl.BlockSpec((R, D), lambda i: (i, 0)),
    )(acc, acc, g, dist, b.reshape(1, D))


def kernel(x, edge_index, edge_weights, W, b):
    ei_flat = edge_index.reshape(2 * E)
    degp = _deg_kernel(ei_flat, edge_weights)
    g, dis = _mmg(x, W, degp.reshape(NC, N).T)
    acc = _agg_kernel(g, ei_flat, edge_weights)
    return _finalize(acc, g, dis, b)
